# Initial kernel scaffold; baseline (speedup 1.0000x reference)
import jax
import jax.numpy as jnp
from jax.experimental import pallas as pl


def kernel(x, edge_index, batch, W1, b1, W2, b2, W3, b3, lin_W, lin_b):
    raise NotImplementedError("write your pallas kernel here")


# speedup vs baseline: 11.1244x; 11.1244x over previous
import functools

import jax
import jax.numpy as jnp
from jax import lax
from jax.experimental import pallas as pl
from jax.experimental.pallas import tpu as pltpu
from jax.experimental.pallas import tpu_sc as plsc

N = 10000
E = 320000
D = 128
C = 10
G = 64

NC = 2
NS = 16
N_PAD = 10240
ROWS_PER_TILE = N_PAD // NS
EDGES_PER_TILE = E // (NC * NS)
K = 80
CHUNKS = EDGES_PER_TILE // K

_mesh = plsc.VectorSubcoreMesh(core_axis_name="c", subcore_axis_name="s")


def _sc_deg_body(dst_hbm, out_hbm, didx, ones_v, zbuf, acc, sem):
    c = lax.axis_index("c")
    s = lax.axis_index("s")

    @pl.loop(0, K, step=16)
    def _(i):
        ones_v[pl.ds(i, 16)] = jnp.ones((16,), jnp.float32)

    @pl.loop(0, ROWS_PER_TILE, step=16)
    def _(i):
        zbuf[pl.ds(i, 16)] = jnp.zeros((16,), jnp.float32)

    pltpu.sync_copy(zbuf, acc.at[pl.ds(s * ROWS_PER_TILE, ROWS_PER_TILE)])
    plsc.subcore_barrier()

    tile_base = (c * NS + s) * EDGES_PER_TILE

    @pl.loop(0, CHUNKS)
    def _(i):
        pltpu.sync_copy(dst_hbm.at[pl.ds(tile_base + i * K, K)], didx)
        pltpu.sync_copy(ones_v, acc.at[didx], add=True)

    plsc.subcore_barrier()
    pltpu.sync_copy(acc.at[pl.ds(s * ROWS_PER_TILE, ROWS_PER_TILE)], zbuf)
    pltpu.sync_copy(zbuf, out_hbm.at[c, pl.ds(s * ROWS_PER_TILE, ROWS_PER_TILE)])


@jax.jit
def _sc_deg(dst):
    return pl.kernel(
        _sc_deg_body,
        out_type=jax.ShapeDtypeStruct((NC, N_PAD), jnp.float32),
        mesh=_mesh,
        scratch_types=[
            pltpu.VMEM((K,), jnp.int32),
            pltpu.VMEM((K,), jnp.float32),
            pltpu.VMEM((ROWS_PER_TILE,), jnp.float32),
            pltpu.VMEM_SHARED((N_PAD,), jnp.float32),
            pltpu.SemaphoreType.DMA,
        ],
    )(dst)


def _sc_edge_body(y_hbm, src_hbm, dst_hbm, out_hbm, sidx, didx, rows, zbuf,
                  acc, sem):
    c = lax.axis_index("c")
    s = lax.axis_index("s")

    @pl.loop(0, K)
    def _(r):
        @pl.loop(0, D, step=16)
        def _(j):
            zbuf[r, pl.ds(j, 16)] = jnp.zeros((16,), jnp.float32)

    @pl.loop(0, ROWS_PER_TILE, step=K)
    def _(r0):
        pltpu.sync_copy(zbuf, acc.at[pl.ds(s * ROWS_PER_TILE + r0, K)])

    plsc.subcore_barrier()

    tile_base = (c * NS + s) * EDGES_PER_TILE

    @pl.loop(0, CHUNKS)
    def _(i):
        base = tile_base + i * K
        pltpu.sync_copy(src_hbm.at[pl.ds(base, K)], sidx)
        pltpu.sync_copy(dst_hbm.at[pl.ds(base, K)], didx)
        pltpu.async_copy(y_hbm.at[sidx], rows, sem).wait()
        pltpu.sync_copy(rows, acc.at[didx], add=True)

    plsc.subcore_barrier()

    @pl.loop(0, ROWS_PER_TILE, step=K)
    def _(r0):
        row0 = s * ROWS_PER_TILE + r0
        pltpu.sync_copy(acc.at[pl.ds(row0, K)], rows)
        pltpu.sync_copy(rows, out_hbm.at[c, pl.ds(row0, K)])


@jax.jit
def _sc_edge(y, src, dst):
    return pl.kernel(
        _sc_edge_body,
        out_type=jax.ShapeDtypeStruct((NC, N_PAD, D), jnp.float32),
        mesh=_mesh,
        scratch_types=[
            pltpu.VMEM((K,), jnp.int32),
            pltpu.VMEM((K,), jnp.int32),
            pltpu.VMEM((K, D), jnp.float32),
            pltpu.VMEM((K, D), jnp.float32),
            pltpu.VMEM_SHARED((N_PAD, D), jnp.float32),
            pltpu.SemaphoreType.DMA,
        ],
    )(y, src, dst)


_BR = 1024


def _tc_matmul_body(x_ref, w_ref, o_ref):
    o_ref[...] = jnp.dot(x_ref[...], w_ref[...],
                         preferred_element_type=jnp.float32)


@jax.jit
def _tc_matmul(x, w):
    grid = (N_PAD // _BR,)
    return pl.pallas_call(
        _tc_matmul_body,
        grid=grid,
        in_specs=[
            pl.BlockSpec((_BR, D), lambda i: (i, 0)),
            pl.BlockSpec((D, D), lambda i: (0, 0)),
        ],
        out_specs=pl.BlockSpec((_BR, D), lambda i: (i, 0)),
        out_shape=jax.ShapeDtypeStruct((N_PAD, D), jnp.float32),
    )(x, w)


def _tc_scale_body(deg0_ref, deg1_ref, xw_ref, dinv_ref, y_ref):
    deg = deg0_ref[...] + deg1_ref[...] + 1.0
    dinv = lax.rsqrt(deg)
    dinv_ref[...] = dinv
    y_ref[...] = xw_ref[...] * dinv


@jax.jit
def _tc_scale(deg0, deg1, xw):
    grid = (N_PAD // _BR,)
    return pl.pallas_call(
        _tc_scale_body,
        grid=grid,
        in_specs=[
            pl.BlockSpec((_BR, 1), lambda i: (i, 0)),
            pl.BlockSpec((_BR, 1), lambda i: (i, 0)),
            pl.BlockSpec((_BR, D), lambda i: (i, 0)),
        ],
        out_specs=[
            pl.BlockSpec((_BR, 1), lambda i: (i, 0)),
            pl.BlockSpec((_BR, D), lambda i: (i, 0)),
        ],
        out_shape=[
            jax.ShapeDtypeStruct((N_PAD, 1), jnp.float32),
            jax.ShapeDtypeStruct((N_PAD, D), jnp.float32),
        ],
    )(deg0, deg1, xw)


def _tc_mid_body(a0_ref, a1_ref, y_ref, dinv_ref, b_ref, w_ref, o_ref):
    dinv = dinv_ref[...]
    h = dinv * (a0_ref[...] + a1_ref[...] + y_ref[...]) + b_ref[...]
    h = jnp.maximum(h, 0.0)
    o_ref[...] = jnp.dot(h, w_ref[...],
                         preferred_element_type=jnp.float32) * dinv


@jax.jit
def _tc_mid(a0, a1, y, dinv, b, w):
    grid = (N_PAD // _BR,)
    return pl.pallas_call(
        _tc_mid_body,
        grid=grid,
        in_specs=[
            pl.BlockSpec((_BR, D), lambda i: (i, 0)),
            pl.BlockSpec((_BR, D), lambda i: (i, 0)),
            pl.BlockSpec((_BR, D), lambda i: (i, 0)),
            pl.BlockSpec((_BR, 1), lambda i: (i, 0)),
            pl.BlockSpec((1, D), lambda i: (0, 0)),
            pl.BlockSpec((D, D), lambda i: (0, 0)),
        ],
        out_specs=pl.BlockSpec((_BR, D), lambda i: (i, 0)),
        out_shape=jax.ShapeDtypeStruct((N_PAD, D), jnp.float32),
    )(a0, a1, y, dinv, b, w)


_BRP = 512


def _tc_post_body(a0_ref, a1_ref, y_ref, dinv_ref, b_ref, batch_ref,
                  linw_ref, linb_ref, o_ref, pool_ref, cnt_ref):
    i = pl.program_id(0)

    @pl.when(i == 0)
    def _():
        pool_ref[...] = jnp.zeros_like(pool_ref)
        cnt_ref[...] = jnp.zeros_like(cnt_ref)

    h = dinv_ref[...] * (a0_ref[...] + a1_ref[...] + y_ref[...]) + b_ref[...]
    gids = lax.broadcasted_iota(jnp.int32, (G, _BRP), 0)
    onehot = (batch_ref[...] == gids).astype(jnp.float32)
    pool_ref[...] += lax.dot_general(
        onehot, h, (((1,), (0,)), ((), ())),
        preferred_element_type=jnp.float32)
    cnt_ref[...] += lax.dot_general(
        onehot, jnp.ones((_BRP, 1), jnp.float32), (((1,), (0,)), ((), ())),
        preferred_element_type=jnp.float32)

    @pl.when(i == (N_PAD // _BRP) - 1)
    def _():
        pooled = pool_ref[...] / jnp.maximum(cnt_ref[...], 1.0)
        o_ref[...] = jnp.dot(pooled, linw_ref[...],
                             preferred_element_type=jnp.float32) + linb_ref[...]


@jax.jit
def _tc_post(a0, a1, y, dinv, b, batch2d, lin_W, lin_b):
    grid = (N_PAD // _BRP,)
    return pl.pallas_call(
        _tc_post_body,
        grid=grid,
        in_specs=[
            pl.BlockSpec((_BRP, D), lambda i: (i, 0)),
            pl.BlockSpec((_BRP, D), lambda i: (i, 0)),
            pl.BlockSpec((_BRP, D), lambda i: (i, 0)),
            pl.BlockSpec((_BRP, 1), lambda i: (i, 0)),
            pl.BlockSpec((1, D), lambda i: (0, 0)),
            pl.BlockSpec((1, _BRP), lambda i: (0, i)),
            pl.BlockSpec((D, C), lambda i: (0, 0)),
            pl.BlockSpec((1, C), lambda i: (0, 0)),
        ],
        out_specs=pl.BlockSpec((G, C), lambda i: (0, 0)),
        out_shape=jax.ShapeDtypeStruct((G, C), jnp.float32),
        scratch_shapes=[
            pltpu.VMEM((G, D), jnp.float32),
            pltpu.VMEM((G, 1), jnp.float32),
        ],
    )(a0, a1, y, dinv, b, batch2d, lin_W, lin_b)


def kernel(x, edge_index, batch, W1, b1, W2, b2, W3, b3, lin_W, lin_b):
    src = edge_index[0].astype(jnp.int32)
    dst = edge_index[1].astype(jnp.int32)
    x_pad = jnp.pad(x, ((0, N_PAD - N), (0, 0)))
    batch2d = jnp.pad(batch.astype(jnp.int32), (0, N_PAD - N),
                      constant_values=G).reshape(1, N_PAD)

    degp = _sc_deg(dst)
    xw1 = _tc_matmul(x_pad, W1)
    deg0 = degp[0].reshape(N_PAD, 1)
    deg1 = degp[1].reshape(N_PAD, 1)
    dinv, y = _tc_scale(deg0, deg1, xw1)

    accp = _sc_edge(y, src, dst)
    y = _tc_mid(accp[0], accp[1], y, dinv, b1.reshape(1, D), W2)
    accp = _sc_edge(y, src, dst)
    y = _tc_mid(accp[0], accp[1], y, dinv, b2.reshape(1, D), W3)
    accp = _sc_edge(y, src, dst)
    return _tc_post(accp[0], accp[1], y, dinv, b3.reshape(1, D), batch2d,
                    lin_W, lin_b.reshape(1, C))
